# baseline (speedup 1.0000x reference)
import functools

import jax
import jax.numpy as jnp
from jax import lax
from jax.experimental import pallas as pl
from jax.experimental.pallas import tpu as pltpu, tpu_sc as plsc

VOCAB = 1000000
D = 64
DP = 128
B = 16384
C = 20
K = 20

NC = 2
NS = 16
NW = NC * NS

ROWS_W = B // NW
R = 16
NCHUNK = ROWS_W // R
IDX_BLK = 80
NBLK = R * C // IDX_BLK


def _cbow_body(tgth_hbm, tgto_hbm, ctxh_hbm, ctxo_hbm, negh_hbm, nego_hbm,
               iv_hbm, ov_hbm, out_hbm,
               idx_ctx_v, off_ctx_v, idx_neg_v, off_neg_v,
               idx_tgt_v, off_tgt_v, ctx_rows_v, orow_v, out_v, sem):
    wid = lax.axis_index("s") * NC + lax.axis_index("c")
    lane = lax.iota(jnp.int32, 16)

    def chunk_body(ch, carry):
        row0 = wid * ROWS_W + ch * R
        off = row0 * C

        pltpu.sync_copy(ctxh_hbm.at[pl.ds(off, R * C)], idx_ctx_v)
        pltpu.sync_copy(ctxo_hbm.at[pl.ds(off * 16, R * C * 16)], off_ctx_v)
        pltpu.sync_copy(negh_hbm.at[pl.ds(off, R * K)], idx_neg_v)
        pltpu.sync_copy(nego_hbm.at[pl.ds(off * 16, R * K * 16)], off_neg_v)
        pltpu.sync_copy(tgth_hbm.at[pl.ds(row0, R)], idx_tgt_v)
        pltpu.sync_copy(tgto_hbm.at[pl.ds(row0 * 16, R * 16)], off_tgt_v)

        copies = []
        for j in range(NBLK):
            copies.append(pltpu.async_copy(
                iv_hbm.at[idx_ctx_v.at[pl.ds(j * IDX_BLK, IDX_BLK)]],
                ctx_rows_v.at[pl.ds(j * IDX_BLK, IDX_BLK)], sem))
            copies.append(pltpu.async_copy(
                ov_hbm.at[idx_neg_v.at[pl.ds(j * IDX_BLK, IDX_BLK)]],
                orow_v.at[pl.ds(j * IDX_BLK, IDX_BLK)], sem))
        copies.append(pltpu.async_copy(
            ov_hbm.at[idx_tgt_v], orow_v.at[pl.ds(R * K, R)], sem))
        for cp in copies:
            cp.wait()

        def row_body(r, rcarry):
            base = r * C

            def half(rows_v, offs_v, prow, row, j):
                hi = offs_v[pl.ds(prow * 16, 16)] != 0
                return jnp.where(hi, rows_v[row, pl.ds(D + j * 16, 16)],
                                 rows_v[row, pl.ds(j * 16, 16)])

            acc = [half(ctx_rows_v, off_ctx_v, base, base, j)
                   for j in range(4)]
            for c in range(1, C):
                for j in range(4):
                    acc[j] = acc[j] + half(
                        ctx_rows_v, off_ctx_v, base + c, base + c, j)
            inv_c = jnp.float32(1.0 / C)
            ctxv = [acc[j] * inv_c for j in range(4)]

            def dot(prow, offs_v, orow):
                v = ctxv[0] * half(orow_v, offs_v, prow, orow, 0)
                for j in range(1, 4):
                    v = v + ctxv[j] * half(orow_v, offs_v, prow, orow, j)
                return jnp.sum(v)

            acc1 = jnp.zeros((16,), jnp.float32)
            acc2 = jnp.where(lane == 4, dot(r, off_tgt_v, R * K + r),
                             jnp.zeros((16,), jnp.float32))
            for k in range(K):
                s = dot(base + k, off_neg_v, base + k)
                if k < 16:
                    acc1 = jnp.where(lane == k, s, acc1)
                else:
                    acc2 = jnp.where(lane == k - 16, s, acc2)
            out_v[r, pl.ds(0, 16)] = acc1
            out_v[r, pl.ds(16, 16)] = acc2
            return rcarry

        lax.fori_loop(0, R, row_body, 0)
        pltpu.sync_copy(out_v, out_hbm.at[pl.ds(row0, R)])
        return carry

    lax.fori_loop(0, NCHUNK, chunk_body, 0)


@functools.partial(jax.jit, static_argnums=())
def _cbow_sc(tgt_h, tgt_o, ctx_h, ctx_o, neg_h, neg_o, input_emb, output_emb):
    iv = input_emb.reshape(VOCAB // 2, DP)
    ov = output_emb.reshape(VOCAB // 2, DP)
    mesh = plsc.VectorSubcoreMesh(core_axis_name="c", subcore_axis_name="s")
    params = pltpu.CompilerParams(needs_layout_passes=False)
    k = pl.kernel(
        _cbow_body,
        mesh=mesh,
        compiler_params=params,
        out_type=jax.ShapeDtypeStruct((B, 32), jnp.float32),
        scratch_types=[
            pltpu.VMEM((R * C,), jnp.int32),
            pltpu.VMEM((R * C * 16,), jnp.int32),
            pltpu.VMEM((R * K,), jnp.int32),
            pltpu.VMEM((R * K * 16,), jnp.int32),
            pltpu.VMEM((R,), jnp.int32),
            pltpu.VMEM((R * 16,), jnp.int32),
            pltpu.VMEM((R * C, DP), jnp.float32),
            pltpu.VMEM((R * K + R, DP), jnp.float32),
            pltpu.VMEM((R, 32), jnp.float32),
            pltpu.SemaphoreType.DMA,
        ],
    )
    return k(tgt_h, tgt_o, ctx_h, ctx_o, neg_h, neg_o, iv, ov)


def kernel(target_ids, context_ids, negative_ids, input_emb, output_emb):
    tgt = target_ids.astype(jnp.int32)
    ctx = context_ids.astype(jnp.int32).reshape(B * C)
    neg = negative_ids.astype(jnp.int32).reshape(B * K)
    rep = lambda a: jnp.repeat(a & 1, 16)
    out = _cbow_sc(tgt >> 1, rep(tgt), ctx >> 1, rep(ctx), neg >> 1, rep(neg),
                   input_emb, output_emb)
    return (out[:, K], out[:, :K])


# speedup vs baseline: 1.5024x; 1.5024x over previous
import functools

import jax
import jax.numpy as jnp
from jax import lax
from jax.experimental import pallas as pl
from jax.experimental.pallas import tpu as pltpu, tpu_sc as plsc

VOCAB = 1000000
D = 64
B = 16384
C = 20
K = 20

NC = 2
NS = 16
NW = NC * NS

ROWS_W = B // NW
R = 32
NCHUNK = ROWS_W // R
IDX_BLK = 128
NBLK = R * C // IDX_BLK


def _cbow_body(tgt_hbm, ctx_hbm, neg_hbm, iemb_hbm, oemb_hbm, out_hbm,
               idx_ctx_v, idx_neg_v, idx_tgt_v,
               ctx_rows_v, orow_v, out_v, sem):
    wid = lax.axis_index("s") * NC + lax.axis_index("c")
    lane = lax.iota(jnp.int32, 16)

    def chunk_body(ch, carry):
        row0 = wid * ROWS_W + ch * R
        off = row0 * C

        pltpu.sync_copy(ctx_hbm.at[pl.ds(off, R * C)], idx_ctx_v)
        pltpu.sync_copy(neg_hbm.at[pl.ds(off, R * K)], idx_neg_v)
        pltpu.sync_copy(tgt_hbm.at[pl.ds(row0, R)], idx_tgt_v)

        copies = []
        for j in range(NBLK):
            copies.append(pltpu.async_copy(
                iemb_hbm.at[idx_ctx_v.at[pl.ds(j * IDX_BLK, IDX_BLK)]],
                ctx_rows_v.at[pl.ds(j * IDX_BLK, IDX_BLK)], sem))
            copies.append(pltpu.async_copy(
                oemb_hbm.at[idx_neg_v.at[pl.ds(j * IDX_BLK, IDX_BLK)]],
                orow_v.at[pl.ds(j * IDX_BLK, IDX_BLK)], sem))
        copies.append(pltpu.async_copy(
            oemb_hbm.at[idx_tgt_v], orow_v.at[pl.ds(R * K, R)], sem))
        for cp in copies:
            cp.wait()

        def row_body(r, rcarry):
            base = r * C

            acc = [ctx_rows_v[base, pl.ds(j * 16, 16)] for j in range(4)]
            for c in range(1, C):
                for j in range(4):
                    acc[j] = acc[j] + ctx_rows_v[base + c, pl.ds(j * 16, 16)]
            inv_c = jnp.float32(1.0 / C)
            ctxv = [acc[j] * inv_c for j in range(4)]

            def dot(orow):
                v = ctxv[0] * orow_v[orow, pl.ds(0, 16)]
                for j in range(1, 4):
                    v = v + ctxv[j] * orow_v[orow, pl.ds(j * 16, 16)]
                return jnp.sum(v)

            acc1 = jnp.zeros((16,), jnp.float32)
            acc2 = jnp.where(lane == 4, dot(R * K + r),
                             jnp.zeros((16,), jnp.float32))
            for k in range(K):
                s = dot(base + k)
                if k < 16:
                    acc1 = jnp.where(lane == k, s, acc1)
                else:
                    acc2 = jnp.where(lane == k - 16, s, acc2)
            out_v[r, pl.ds(0, 16)] = acc1
            out_v[r, pl.ds(16, 16)] = acc2
            return rcarry

        lax.fori_loop(0, R, row_body, 0)
        pltpu.sync_copy(out_v, out_hbm.at[pl.ds(row0, R)])
        return carry

    lax.fori_loop(0, NCHUNK, chunk_body, 0)


@functools.partial(jax.jit, static_argnums=())
def _cbow_sc(tgt, ctx, neg, input_emb, output_emb):
    mesh = plsc.VectorSubcoreMesh(core_axis_name="c", subcore_axis_name="s")
    params = pltpu.CompilerParams(needs_layout_passes=False,
                                  use_tc_tiling_on_sc=False)
    k = pl.kernel(
        _cbow_body,
        mesh=mesh,
        compiler_params=params,
        out_type=jax.ShapeDtypeStruct((B, 32), jnp.float32),
        scratch_types=[
            pltpu.VMEM((R * C,), jnp.int32),
            pltpu.VMEM((R * K,), jnp.int32),
            pltpu.VMEM((R,), jnp.int32),
            pltpu.VMEM((R * C, D), jnp.float32),
            pltpu.VMEM((R * K + R, D), jnp.float32),
            pltpu.VMEM((R, 32), jnp.float32),
            pltpu.SemaphoreType.DMA,
        ],
    )
    return k(tgt, ctx, neg, input_emb, output_emb)


def kernel(target_ids, context_ids, negative_ids, input_emb, output_emb):
    tgt = target_ids.astype(jnp.int32)
    ctx = context_ids.astype(jnp.int32).reshape(B * C)
    neg = negative_ids.astype(jnp.int32).reshape(B * K)
    out = _cbow_sc(tgt, ctx, neg, input_emb, output_emb)
    return (out[:, K], out[:, :K])
